# Initial kernel scaffold; baseline (speedup 1.0000x reference)
import jax
import jax.numpy as jnp
from jax.experimental import pallas as pl


def kernel(x, q_input, Wq, Wk, gamma, beta, Ww):
    raise NotImplementedError("write your pallas kernel here")


# speedup vs baseline: 1.0213x; 1.0213x over previous
import functools

import jax
import jax.numpy as jnp
from jax.experimental import pallas as pl
from jax.experimental.pallas import tpu as pltpu

B, S, DM, QIN = 1, 2048, 1024, 1024
H, D, TOPK = 16, 64, 512
QBLK = 256
SCALE = D ** -0.5
WSCALE = H ** -0.5


def _prep_kernel(x_ref, wkT_ref, gamma_ref, beta_ref, wwT_ref, kT_ref, w_ref):
    x = x_ref[...]
    k = jnp.dot(x, wkT_ref[...], preferred_element_type=jnp.float32)
    mu = jnp.mean(k, axis=-1, keepdims=True)
    var = jnp.mean((k - mu) ** 2, axis=-1, keepdims=True)
    k = (k - mu) / jnp.sqrt(var + 1e-5) * gamma_ref[...] + beta_ref[...]
    kT_ref[...] = k.T
    w_ref[...] = jnp.dot(x, wwT_ref[...], preferred_element_type=jnp.float32) * WSCALE


def _scores_kernel(q_in_ref, wqT_ref, kT_ref, w_ref, s_ref):
    q = jnp.dot(q_in_ref[...], wqT_ref[...], preferred_element_type=jnp.float32)
    w = w_ref[...]
    acc = jnp.zeros((QBLK, S), dtype=jnp.float32)
    for h in range(H):
        qh = q[:, h * D:(h + 1) * D]
        sh = jnp.dot(qh, kT_ref[...], preferred_element_type=jnp.float32)
        acc = acc + sh * (w[:, h:h + 1] * SCALE)
    s_ref[...] = acc


def _layernorm_host(v, gamma, beta, eps=1e-5):
    mu = jnp.mean(v, axis=-1, keepdims=True)
    var = jnp.var(v, axis=-1, keepdims=True)
    return (v - mu) / jnp.sqrt(var + eps) * gamma + beta


def kernel(x, q_input, Wq, Wk, gamma, beta, Ww):
    x2 = x.reshape(S, DM)
    q2 = q_input.reshape(S, QIN)
    k = _layernorm_host(x2 @ Wk.T, gamma, beta)
    kT = k.T
    w = (x2 @ Ww.T) * WSCALE

    scores = pl.pallas_call(
        _scores_kernel,
        grid=(S // QBLK,),
        in_specs=[
            pl.BlockSpec((QBLK, QIN), lambda i: (i, 0)),
            pl.BlockSpec((QIN, H * D), lambda i: (0, 0)),
            pl.BlockSpec((D, S), lambda i: (0, 0)),
            pl.BlockSpec((QBLK, H), lambda i: (i, 0)),
        ],
        out_specs=pl.BlockSpec((QBLK, S), lambda i: (i, 0)),
        out_shape=jax.ShapeDtypeStruct((S, S), jnp.float32),
    )(q2, Wq.T, kT, w)

    _, idx = jax.lax.top_k(scores, TOPK)
    return idx.reshape(B, S, TOPK)
